# Initial kernel scaffold; baseline (speedup 1.0000x reference)
import jax
import jax.numpy as jnp
from jax.experimental import pallas as pl


def kernel(emb_weight, pos):
    raise NotImplementedError("write your pallas kernel here")


# speedup vs baseline: 1.5503x; 1.5503x over previous
import functools

import jax
import jax.numpy as jnp
from jax import lax
from jax.experimental import pallas as pl
from jax.experimental.pallas import tpu as pltpu
from jax.experimental.pallas import tpu_sc as plsc

MAXLEN = 8192
D_MODEL = 1024
NUM_CORES = 2
NUM_SUBCORES = 16
NUM_WORKERS = NUM_CORES * NUM_SUBCORES
ROWS_PER_WORKER = MAXLEN // NUM_WORKERS
CHUNK = 32
NUM_CHUNKS = ROWS_PER_WORKER // CHUNK

_MESH = plsc.VectorSubcoreMesh(core_axis_name="c", subcore_axis_name="s")


@functools.partial(
    pl.kernel,
    mesh=_MESH,
    out_type=jax.ShapeDtypeStruct((MAXLEN, D_MODEL), jnp.float32),
    scratch_types=[
        pltpu.VMEM((ROWS_PER_WORKER,), jnp.int32),
        pltpu.VMEM((CHUNK, D_MODEL), jnp.float32),
        pltpu.VMEM((CHUNK, D_MODEL), jnp.float32),
        pltpu.SemaphoreType.DMA,
        pltpu.SemaphoreType.DMA,
        pltpu.SemaphoreType.DMA,
        pltpu.SemaphoreType.DMA,
    ],
)
def _emb_lookup(table_hbm, idx_hbm, out_hbm, idx_v, buf0, buf1,
                gsem0, gsem1, ssem0, ssem1):
    wid = lax.axis_index("s") * NUM_CORES + lax.axis_index("c")
    base = wid * ROWS_PER_WORKER

    pltpu.sync_copy(idx_hbm.at[pl.ds(base, ROWS_PER_WORKER)], idx_v)

    bufs = (buf0, buf1)
    gsems = (gsem0, gsem1)
    ssems = (ssem0, ssem1)
    gathers = [None, None]
    scatters = [None, None]

    def start_gather(c):
        b = c % 2
        idx_chunk = idx_v.at[pl.ds(c * CHUNK, CHUNK)]
        gathers[b] = pltpu.async_copy(table_hbm.at[idx_chunk], bufs[b], gsems[b])

    start_gather(0)
    for c in range(NUM_CHUNKS):
        b = c % 2
        if c + 1 < NUM_CHUNKS:
            nb = (c + 1) % 2
            if scatters[nb] is not None:
                scatters[nb].wait()
                scatters[nb] = None
            start_gather(c + 1)
        gathers[b].wait()
        out_slice = out_hbm.at[pl.ds(base + c * CHUNK, CHUNK)]
        scatters[b] = pltpu.async_copy(bufs[b], out_slice, ssems[b])
    for b in range(2):
        if scatters[b] is not None:
            scatters[b].wait()


def kernel(emb_weight, pos):
    return _emb_lookup(emb_weight, pos.astype(jnp.int32))


# speedup vs baseline: 1.5758x; 1.0165x over previous
import functools

import jax
import jax.numpy as jnp
from jax import lax
from jax.experimental import pallas as pl
from jax.experimental.pallas import tpu as pltpu
from jax.experimental.pallas import tpu_sc as plsc

MAXLEN = 8192
D_MODEL = 1024
NUM_CORES = 2
NUM_SUBCORES = 16
NUM_WORKERS = NUM_CORES * NUM_SUBCORES
ROWS_PER_WORKER = MAXLEN // NUM_WORKERS
CHUNK = 32
NUM_CHUNKS = ROWS_PER_WORKER // CHUNK
NBUF = 3

_MESH = plsc.VectorSubcoreMesh(core_axis_name="c", subcore_axis_name="s")


@functools.partial(
    pl.kernel,
    mesh=_MESH,
    out_type=jax.ShapeDtypeStruct((MAXLEN, D_MODEL), jnp.float32),
    scratch_types=[
        pltpu.VMEM((ROWS_PER_WORKER,), jnp.int32),
    ]
    + [pltpu.VMEM((CHUNK, D_MODEL), jnp.float32)] * NBUF
    + [pltpu.SemaphoreType.DMA] * (2 * NBUF),
)
def _emb_lookup(table_hbm, idx_hbm, out_hbm, idx_v, *bufs_and_sems):
    bufs = bufs_and_sems[:NBUF]
    gsems = bufs_and_sems[NBUF:2 * NBUF]
    ssems = bufs_and_sems[2 * NBUF:]

    wid = lax.axis_index("s") * NUM_CORES + lax.axis_index("c")
    base = wid * ROWS_PER_WORKER

    pltpu.sync_copy(idx_hbm.at[pl.ds(base, ROWS_PER_WORKER)], idx_v)

    gathers = [None] * NBUF
    scatters = [None] * NBUF

    def start_gather(c):
        b = c % NBUF
        idx_chunk = idx_v.at[pl.ds(c * CHUNK, CHUNK)]
        gathers[b] = pltpu.async_copy(table_hbm.at[idx_chunk], bufs[b], gsems[b])

    for c in range(min(NBUF - 1, NUM_CHUNKS)):
        start_gather(c)
    for c in range(NUM_CHUNKS):
        b = c % NBUF
        nxt = c + NBUF - 1
        if nxt < NUM_CHUNKS:
            nb = nxt % NBUF
            if scatters[nb] is not None:
                scatters[nb].wait()
                scatters[nb] = None
            start_gather(nxt)
        gathers[b].wait()
        out_slice = out_hbm.at[pl.ds(base + c * CHUNK, CHUNK)]
        scatters[b] = pltpu.async_copy(bufs[b], out_slice, ssems[b])
    for b in range(NBUF):
        if scatters[b] is not None:
            scatters[b].wait()


def kernel(emb_weight, pos):
    return _emb_lookup(emb_weight, pos.astype(jnp.int32))


# speedup vs baseline: 1.5852x; 1.0059x over previous
import functools

import jax
import jax.numpy as jnp
from jax import lax
from jax.experimental import pallas as pl
from jax.experimental.pallas import tpu as pltpu
from jax.experimental.pallas import tpu_sc as plsc

MAXLEN = 8192
D_MODEL = 1024
NUM_CORES = 2
NUM_SUBCORES = 16
NUM_WORKERS = NUM_CORES * NUM_SUBCORES
ROWS_PER_WORKER = MAXLEN // NUM_WORKERS
CHUNK = 56
NBUF = 2
CHUNK_OFFS = [56 * i for i in range(ROWS_PER_WORKER // CHUNK)] + [224]
CHUNK_LENS = [56] * (ROWS_PER_WORKER // CHUNK) + [32]
NUM_CHUNKS = len(CHUNK_OFFS)

_MESH = plsc.VectorSubcoreMesh(core_axis_name="c", subcore_axis_name="s")


@functools.partial(
    pl.kernel,
    mesh=_MESH,
    out_type=jax.ShapeDtypeStruct((MAXLEN, D_MODEL), jnp.float32),
    scratch_types=[
        pltpu.VMEM((ROWS_PER_WORKER,), jnp.int32),
    ]
    + [pltpu.VMEM((CHUNK, D_MODEL), jnp.float32)] * NBUF
    + [pltpu.SemaphoreType.DMA] * (2 * NBUF),
)
def _emb_lookup(table_hbm, idx_hbm, out_hbm, idx_v, *bufs_and_sems):
    bufs = bufs_and_sems[:NBUF]
    gsems = bufs_and_sems[NBUF:2 * NBUF]
    ssems = bufs_and_sems[2 * NBUF:]

    wid = lax.axis_index("s") * NUM_CORES + lax.axis_index("c")
    base = wid * ROWS_PER_WORKER

    pltpu.sync_copy(idx_hbm.at[pl.ds(base, ROWS_PER_WORKER)], idx_v)

    gathers = [None] * NBUF
    scatters = [None] * NBUF

    def start_gather(c):
        b = c % NBUF
        off, n = CHUNK_OFFS[c], CHUNK_LENS[c]
        idx_chunk = idx_v.at[pl.ds(off, n)]
        dst = bufs[b] if n == CHUNK else bufs[b].at[pl.ds(0, n)]
        gathers[b] = pltpu.async_copy(table_hbm.at[idx_chunk], dst, gsems[b])

    for c in range(min(NBUF - 1, NUM_CHUNKS)):
        start_gather(c)
    for c in range(NUM_CHUNKS):
        b = c % NBUF
        nxt = c + NBUF - 1
        if nxt < NUM_CHUNKS:
            nb = nxt % NBUF
            if scatters[nb] is not None:
                scatters[nb].wait()
                scatters[nb] = None
            start_gather(nxt)
        gathers[b].wait()
        off, n = CHUNK_OFFS[c], CHUNK_LENS[c]
        src = bufs[b] if n == CHUNK else bufs[b].at[pl.ds(0, n)]
        out_slice = out_hbm.at[pl.ds(base + off, n)]
        scatters[b] = pltpu.async_copy(src, out_slice, ssems[b])
    for b in range(NBUF):
        if scatters[b] is not None:
            scatters[b].wait()


def kernel(emb_weight, pos):
    return _emb_lookup(emb_weight, pos.astype(jnp.int32))
